# baseline (speedup 1.0000x reference)
import functools

import jax
import jax.numpy as jnp
from jax import lax
from jax.experimental import pallas as pl
from jax.experimental.pallas import tpu as pltpu
from jax.experimental.pallas import tpu_sc as plsc

NUM_WORKERS = 32
CHUNK = 512


def _emb_body(tid_hbm, table_hbm, out_hbm, idx2d, idxt, rows_v, g0, g1,
              w0, w1, *, h_dim):
    wid = lax.axis_index("s") * 2 + lax.axis_index("c")
    b0 = wid * CHUNK
    pltpu.sync_copy(tid_hbm.at[pl.ds(b0, CHUNK), :], idx2d)
    gsem = (g0, g1)
    wsem = (w0, w1)
    iota = lax.iota(jnp.int32, 16)

    @plsc.parallel_loop(0, CHUNK // 16, step=1, unroll=2)
    def tk(k):
        base = k * 16
        ridx = base + iota
        for h in range(h_dim):
            v = plsc.load_gather(idx2d, [ridx, jnp.full((16,), h, jnp.int32)])
            idxt[h, pl.ds(base, 16)] = v

    def g_start(h, b):
        pltpu.make_async_copy(table_hbm.at[idxt.at[h]], rows_v.at[b],
                              gsem[b]).start()

    def g_wait(b):
        pltpu.make_async_copy(table_hbm.at[idxt.at[0]], rows_v.at[b],
                              gsem[b]).wait()

    def w_start(h, b):
        pltpu.make_async_copy(rows_v.at[b], out_hbm.at[pl.ds(b0, CHUNK), h, :],
                              wsem[b]).start()

    def w_wait(b):
        pltpu.make_async_copy(rows_v.at[b], out_hbm.at[pl.ds(0, CHUNK), 0, :],
                              wsem[b]).wait()

    g_start(0, 0)
    g_start(1, 1)
    g_wait(0)
    w_start(0, 0)

    def pair(go, carry):
        u = 2 * go + 1
        g_wait(1)
        w_start(u, 1)
        w_wait(0)
        g_start(u + 1, 0)
        g_wait(0)
        w_start(u + 1, 0)
        w_wait(1)
        g_start(u + 2, 1)
        return carry

    lax.fori_loop(0, (h_dim - 2) // 2, pair, 0)

    g_wait(1)
    w_start(h_dim - 1, 1)
    w_wait(0)
    w_wait(1)


def kernel(token_id, table):
    B, H = token_id.shape
    V, D = table.shape

    mesh = plsc.VectorSubcoreMesh(core_axis_name="c", subcore_axis_name="s")
    emb = functools.partial(
        pl.kernel,
        mesh=mesh,
        out_type=jax.ShapeDtypeStruct((B, H, D), jnp.float32),
        scratch_types=[
            pltpu.VMEM((CHUNK, H), jnp.int32),
            pltpu.VMEM((H, CHUNK), jnp.int32),
            pltpu.VMEM((2, CHUNK, D), jnp.float32),
            pltpu.SemaphoreType.DMA,
            pltpu.SemaphoreType.DMA,
            pltpu.SemaphoreType.DMA,
            pltpu.SemaphoreType.DMA,
        ],
        compiler_params=pltpu.CompilerParams(use_tc_tiling_on_sc=False,
                                             needs_layout_passes=False),
    )(functools.partial(_emb_body, h_dim=H))

    return emb(token_id, table)


# speedup vs baseline: 1.0075x; 1.0075x over previous
import functools

import jax
import jax.numpy as jnp
from jax import lax
from jax.experimental import pallas as pl
from jax.experimental.pallas import tpu as pltpu
from jax.experimental.pallas import tpu_sc as plsc

NUM_WORKERS = 32
CHUNK = 512
HPAD = 8


def _emb_body(tid_hbm, table_hbm, out_hbm, idx_v, rows_v, g0, g1, w0, w1,
              *, h_dim):
    wid = lax.axis_index("s") * 2 + lax.axis_index("c")
    b0 = wid * CHUNK
    pltpu.sync_copy(tid_hbm.at[pl.ds(0, h_dim), pl.ds(b0, CHUNK)], idx_v)
    gsem = (g0, g1)
    wsem = (w0, w1)

    def g_start(h, b):
        pltpu.make_async_copy(table_hbm.at[idx_v.at[h]], rows_v.at[b],
                              gsem[b]).start()

    def g_wait(b):
        pltpu.make_async_copy(table_hbm.at[idx_v.at[0]], rows_v.at[b],
                              gsem[b]).wait()

    def w_start(h, b):
        pltpu.make_async_copy(rows_v.at[b], out_hbm.at[pl.ds(b0, CHUNK), h, :],
                              wsem[b]).start()

    def w_wait(b):
        pltpu.make_async_copy(rows_v.at[b], out_hbm.at[pl.ds(0, CHUNK), 0, :],
                              wsem[b]).wait()

    g_start(0, 0)
    g_start(1, 1)
    g_wait(0)
    w_start(0, 0)

    def pair(go, carry):
        u = 2 * go + 1
        g_wait(1)
        w_start(u, 1)
        w_wait(0)
        g_start(u + 1, 0)
        g_wait(0)
        w_start(u + 1, 0)
        w_wait(1)
        g_start(u + 2, 1)
        return carry

    lax.fori_loop(0, (h_dim - 2) // 2, pair, 0)

    g_wait(1)
    w_start(h_dim - 1, 1)
    w_wait(0)
    w_wait(1)


def kernel(token_id, table):
    B, H = token_id.shape
    V, D = table.shape
    hp = (H + HPAD - 1) // HPAD * HPAD
    tid_t = jnp.pad(token_id, ((0, 0), (0, hp - H))).T.astype(jnp.int32)

    mesh = plsc.VectorSubcoreMesh(core_axis_name="c", subcore_axis_name="s")
    emb = functools.partial(
        pl.kernel,
        mesh=mesh,
        out_type=jax.ShapeDtypeStruct((B, H, D), jnp.float32),
        scratch_types=[
            pltpu.VMEM((H, CHUNK), jnp.int32),
            pltpu.VMEM((2, CHUNK, D), jnp.float32),
            pltpu.SemaphoreType.DMA,
            pltpu.SemaphoreType.DMA,
            pltpu.SemaphoreType.DMA,
            pltpu.SemaphoreType.DMA,
        ],
        compiler_params=pltpu.CompilerParams(use_tc_tiling_on_sc=False,
                                             needs_layout_passes=False),
    )(functools.partial(_emb_body, h_dim=H))

    return emb(tid_t, table)
